# Initial kernel scaffold; baseline (speedup 1.0000x reference)
import jax
import jax.numpy as jnp
from jax.experimental import pallas as pl


def kernel(org_img, params, color_map_control):
    raise NotImplementedError("write your pallas kernel here")


# speedup vs baseline: 541.9719x; 541.9719x over previous
import dataclasses
import functools

import jax
import jax.numpy as jnp
from jax.experimental import pallas as pl
from jax.experimental.pallas import tpu as pltpu
from jax.experimental.pallas import tpu_sc as plsc

CP = 64
NCHAN = 96
NPIX = 512 * 512
CHUNK = 4096
LANES = 16


def _sc_call(cmc2, par2, img2):
    mesh = plsc.VectorSubcoreMesh(core_axis_name="c", subcore_axis_name="s")
    cp_params = pltpu.CompilerParams()
    if "needs_layout_passes" in pltpu.CompilerParams.__dataclass_fields__:
        cp_params = dataclasses.replace(cp_params, needs_layout_passes=False)

    @functools.partial(
        pl.kernel,
        out_type=jax.ShapeDtypeStruct((NCHAN, NPIX), jnp.float32),
        mesh=mesh,
        scratch_types=[pltpu.VMEM((CP,), jnp.float32)],
        compiler_params=cp_params,
    )
    def run(cmc_hbm, par_hbm, img_hbm, out_hbm, ytab_ref):
        def body(cmc_v, par_v, img_v, out_v):
            for t in range(CP // LANES):
                sl = pl.ds(t * LANES, LANES)
                ytab_ref[sl] = cmc_v[0, sl] + par_v[0, sl] * 0.04

            @pl.loop(0, CHUNK, step=LANES)
            def _(c0):
                sl = pl.ds(c0, LANES)
                x = img_v[0, sl]
                v = x * 63.0
                i = jnp.minimum(v.astype(jnp.int32), 62)
                coeff = v - i.astype(jnp.float32)
                y0 = plsc.load_gather(ytab_ref, [i])
                y1 = plsc.load_gather(ytab_ref, [i + 1])
                out_v[0, sl] = (1.0 - coeff) * y0 + coeff * y1

        pltpu.emit_pipeline(
            body,
            grid=(NCHAN, NPIX // CHUNK),
            in_specs=[
                pl.BlockSpec((1, CP), lambda i, j: (i, 0)),
                pl.BlockSpec((1, CP), lambda i, j: (i, 0)),
                pl.BlockSpec((1, CHUNK), lambda i, j: (i, j)),
            ],
            out_specs=[pl.BlockSpec((1, CHUNK), lambda i, j: (i, j))],
            core_axis_name=("c", "s"),
            dimension_semantics=(pltpu.PARALLEL, pltpu.PARALLEL),
        )(cmc_hbm, par_hbm, img_hbm, out_hbm)

    return run(cmc2, par2, img2)


def kernel(org_img, params, color_map_control):
    N, C, H, W = org_img.shape
    img2 = org_img.reshape(NCHAN, NPIX)
    cmc2 = color_map_control.reshape(NCHAN, CP)
    par2 = params.reshape(NCHAN, CP)
    out = _sc_call(cmc2, par2, img2)
    return out.reshape(N, C, H, W)


# speedup vs baseline: 545.5519x; 1.0066x over previous
import dataclasses
import functools

import jax
import jax.numpy as jnp
from jax.experimental import pallas as pl
from jax.experimental.pallas import tpu as pltpu
from jax.experimental.pallas import tpu_sc as plsc

CP = 64
NCHAN = 96
NPIX = 512 * 512
CHUNK = 16384
LANES = 16


def _sc_call(cmc2, par2, img2):
    mesh = plsc.VectorSubcoreMesh(core_axis_name="c", subcore_axis_name="s")
    cp_params = pltpu.CompilerParams()
    if "needs_layout_passes" in pltpu.CompilerParams.__dataclass_fields__:
        cp_params = dataclasses.replace(cp_params, needs_layout_passes=False)

    @functools.partial(
        pl.kernel,
        out_type=jax.ShapeDtypeStruct((NCHAN, NPIX), jnp.float32),
        mesh=mesh,
        scratch_types=[pltpu.VMEM((CP,), jnp.float32)],
        compiler_params=cp_params,
    )
    def run(cmc_hbm, par_hbm, img_hbm, out_hbm, ytab_ref):
        def body(cmc_v, par_v, img_v, out_v):
            for t in range(CP // LANES):
                sl = pl.ds(t * LANES, LANES)
                ytab_ref[sl] = cmc_v[0, sl] + par_v[0, sl] * 0.04

            @pl.loop(0, CHUNK, step=LANES)
            def _(c0):
                sl = pl.ds(c0, LANES)
                x = img_v[0, sl]
                v = x * 63.0
                i = jnp.minimum(v.astype(jnp.int32), 62)
                coeff = v - i.astype(jnp.float32)
                y0 = plsc.load_gather(ytab_ref, [i])
                y1 = plsc.load_gather(ytab_ref, [i + 1])
                out_v[0, sl] = (1.0 - coeff) * y0 + coeff * y1

        pltpu.emit_pipeline(
            body,
            grid=(NCHAN, NPIX // CHUNK),
            in_specs=[
                pl.BlockSpec((1, CP), lambda i, j: (i, 0)),
                pl.BlockSpec((1, CP), lambda i, j: (i, 0)),
                pl.BlockSpec((1, CHUNK), lambda i, j: (i, j)),
            ],
            out_specs=[pl.BlockSpec((1, CHUNK), lambda i, j: (i, j))],
            core_axis_name=("c", "s"),
            dimension_semantics=(pltpu.PARALLEL, pltpu.PARALLEL),
        )(cmc_hbm, par_hbm, img_hbm, out_hbm)

    return run(cmc2, par2, img2)


def kernel(org_img, params, color_map_control):
    N, C, H, W = org_img.shape
    img2 = org_img.reshape(NCHAN, NPIX)
    cmc2 = color_map_control.reshape(NCHAN, CP)
    par2 = params.reshape(NCHAN, CP)
    out = _sc_call(cmc2, par2, img2)
    return out.reshape(N, C, H, W)


# speedup vs baseline: 1680.0240x; 3.0795x over previous
import dataclasses
import functools

import jax
import jax.numpy as jnp
from jax.experimental import pallas as pl
from jax.experimental.pallas import tpu as pltpu
from jax.experimental.pallas import tpu_sc as plsc

CP = 64
NCHAN = 96
NPIX = 512 * 512
CHUNK = 16384
LANES = 16


def _sc_call(cmc2, par2, img2):
    mesh = plsc.VectorSubcoreMesh(core_axis_name="c", subcore_axis_name="s")
    cp_params = pltpu.CompilerParams()
    if "needs_layout_passes" in pltpu.CompilerParams.__dataclass_fields__:
        cp_params = dataclasses.replace(cp_params, needs_layout_passes=False)

    @functools.partial(
        pl.kernel,
        out_type=jax.ShapeDtypeStruct((NCHAN, NPIX), jnp.float32),
        mesh=mesh,
        scratch_types=[pltpu.VMEM((CP,), jnp.float32)],
        compiler_params=cp_params,
    )
    def run(cmc_hbm, par_hbm, img_hbm, out_hbm, ytab_ref):
        def body(cmc_v, par_v, img_v, out_v):
            for t in range(CP // LANES):
                sl = pl.ds(t * LANES, LANES)
                ytab_ref[sl] = cmc_v[0, sl] + par_v[0, sl] * 0.04

            @plsc.parallel_loop(0, CHUNK, step=LANES, unroll=8)
            def _(c0):
                sl = pl.ds(c0, LANES)
                x = img_v[0, sl]
                v = x * 63.0
                i = jnp.minimum(v.astype(jnp.int32), 62)
                coeff = v - i.astype(jnp.float32)
                y0 = plsc.load_gather(ytab_ref, [i])
                y1 = plsc.load_gather(ytab_ref, [i + 1])
                out_v[0, sl] = (1.0 - coeff) * y0 + coeff * y1

        pltpu.emit_pipeline(
            body,
            grid=(NCHAN, NPIX // CHUNK),
            in_specs=[
                pl.BlockSpec((1, CP), lambda i, j: (i, 0)),
                pl.BlockSpec((1, CP), lambda i, j: (i, 0)),
                pl.BlockSpec((1, CHUNK), lambda i, j: (i, j)),
            ],
            out_specs=[pl.BlockSpec((1, CHUNK), lambda i, j: (i, j))],
            core_axis_name=("c", "s"),
            dimension_semantics=(pltpu.PARALLEL, pltpu.PARALLEL),
        )(cmc_hbm, par_hbm, img_hbm, out_hbm)

    return run(cmc2, par2, img2)


def kernel(org_img, params, color_map_control):
    N, C, H, W = org_img.shape
    img2 = org_img.reshape(NCHAN, NPIX)
    cmc2 = color_map_control.reshape(NCHAN, CP)
    par2 = params.reshape(NCHAN, CP)
    out = _sc_call(cmc2, par2, img2)
    return out.reshape(N, C, H, W)
